# baseline (speedup 1.0000x reference)
import functools

import jax
import jax.numpy as jnp
from jax import lax
from jax.experimental import pallas as pl
from jax.experimental.pallas import tpu as pltpu
from jax.experimental.pallas import tpu_sc as plsc

EMB = 64
HID = 256
BATCH = 16384

NC = 2
NS = 16
NW = NC * NS
CHUNK = 128
B_PER_W = BATCH // NW
K = B_PER_W // CHUNK
NROWS = BATCH // CHUNK


def _gather_body(uids_hbm, iids_hbm, uemb_hbm, iemb_hbm, uout_hbm, iout_hbm,
                 uidx_v, iidx_v, urows_v, irows_v, sem):
    wid = lax.axis_index("s") * NC + lax.axis_index("c")
    base = wid * K
    pltpu.sync_copy(uids_hbm.at[pl.ds(base, K)], uidx_v)
    pltpu.sync_copy(iids_hbm.at[pl.ds(base, K)], iidx_v)
    copies = []
    for j in range(K):
        copies.append(pltpu.async_copy(uemb_hbm.at[uidx_v.at[j]], urows_v.at[j], sem))
        copies.append(pltpu.async_copy(iemb_hbm.at[iidx_v.at[j]], irows_v.at[j], sem))
    for c in copies:
        c.wait()
    pltpu.sync_copy(urows_v, uout_hbm.at[pl.ds(base, K)])
    pltpu.sync_copy(irows_v, iout_hbm.at[pl.ds(base, K)])


@jax.jit
def _gather(user_ids, item_ids, user_emb, item_emb):
    mesh = plsc.VectorSubcoreMesh(core_axis_name="c", subcore_axis_name="s")
    fn = functools.partial(
        pl.kernel,
        mesh=mesh,
        out_type=[
            jax.ShapeDtypeStruct((NROWS, CHUNK, EMB), jnp.float32),
            jax.ShapeDtypeStruct((NROWS, CHUNK, EMB), jnp.float32),
        ],
        scratch_types=[
            pltpu.VMEM((K, CHUNK), jnp.int32),
            pltpu.VMEM((K, CHUNK), jnp.int32),
            pltpu.VMEM((K, CHUNK, EMB), jnp.float32),
            pltpu.VMEM((K, CHUNK, EMB), jnp.float32),
            pltpu.SemaphoreType.DMA,
        ],
        compiler_params=pltpu.CompilerParams(use_tc_tiling_on_sc=False),
    )(_gather_body)
    uout, iout = fn(
        user_ids.reshape(NROWS, CHUNK),
        item_ids.reshape(NROWS, CHUNK),
        user_emb,
        item_emb,
    )
    return uout.reshape(BATCH, EMB), iout.reshape(BATCH, EMB)


BS = 2048


def _mlp_body(u_ref, i_ref, w1u_ref, w1i_ref, b1_ref, w2t_ref, b2_ref, out_ref):
    x = jnp.dot(u_ref[...], w1u_ref[...], preferred_element_type=jnp.float32)
    x = x + jnp.dot(i_ref[...], w1i_ref[...], preferred_element_type=jnp.float32)
    x = jnp.maximum(x + b1_ref[...], 0.0)
    y = jnp.sum(x * w2t_ref[...], axis=1, keepdims=True)
    out_ref[...] = y + b2_ref[...]


@jax.jit
def _mlp(u, i, w1u, w1i, b1, w2t, b2):
    grid = (BATCH // BS,)
    return pl.pallas_call(
        _mlp_body,
        grid=grid,
        in_specs=[
            pl.BlockSpec((BS, EMB), lambda g: (g, 0)),
            pl.BlockSpec((BS, EMB), lambda g: (g, 0)),
            pl.BlockSpec((EMB, HID), lambda g: (0, 0)),
            pl.BlockSpec((EMB, HID), lambda g: (0, 0)),
            pl.BlockSpec((1, HID), lambda g: (0, 0)),
            pl.BlockSpec((1, HID), lambda g: (0, 0)),
            pl.BlockSpec((1, 1), lambda g: (0, 0)),
        ],
        out_specs=pl.BlockSpec((BS, 1), lambda g: (g, 0)),
        out_shape=jax.ShapeDtypeStruct((BATCH, 1), jnp.float32),
    )(u, i, w1u, w1i, b1, w2t, b2)


def kernel(user_ids, item_ids, user_emb, item_emb, W1, b1, W2, b2):
    uids = user_ids.astype(jnp.int32)
    iids = item_ids.astype(jnp.int32)
    u, i = _gather(uids, iids, user_emb, item_emb)
    w1u = W1[:EMB]
    w1i = W1[EMB:]
    return _mlp(u, i, w1u, w1i, b1.reshape(1, HID), W2.reshape(1, HID),
                b2.reshape(1, 1))


# speedup vs baseline: 1.0268x; 1.0268x over previous
import functools

import jax
import jax.numpy as jnp
from jax import lax
from jax.experimental import pallas as pl
from jax.experimental.pallas import tpu as pltpu
from jax.experimental.pallas import tpu_sc as plsc

EMB = 64
HID = 256
BATCH = 16384

NC = 2
NS = 16
NW = NC * NS
CHUNK = 128
B_PER_W = BATCH // NW
K = B_PER_W // CHUNK


def _gather_body(uids_hbm, iids_hbm, uemb_hbm, iemb_hbm, comb_hbm,
                 uidx_v, iidx_v, urows_v, irows_v, sem):
    wid = lax.axis_index("s") * NC + lax.axis_index("c")
    base = wid * B_PER_W
    for j in range(K):
        pltpu.sync_copy(uids_hbm.at[pl.ds(base + j * CHUNK, CHUNK)], uidx_v.at[j])
        pltpu.sync_copy(iids_hbm.at[pl.ds(base + j * CHUNK, CHUNK)], iidx_v.at[j])
    copies = []
    for j in range(K):
        copies.append(pltpu.async_copy(
            uemb_hbm.at[uidx_v.at[j]], urows_v.at[j], sem))
        copies.append(pltpu.async_copy(
            iemb_hbm.at[iidx_v.at[j]], irows_v.at[j], sem))
    for c in copies:
        c.wait()
    for j in range(K):
        row0 = base + j * CHUNK
        pltpu.sync_copy(urows_v.at[j],
                        comb_hbm.at[pl.ds(row0, CHUNK), pl.ds(0, EMB)])
        pltpu.sync_copy(irows_v.at[j],
                        comb_hbm.at[pl.ds(row0, CHUNK), pl.ds(EMB, EMB)])


@jax.jit
def _gather(user_ids, item_ids, user_emb, item_emb):
    mesh = plsc.VectorSubcoreMesh(core_axis_name="c", subcore_axis_name="s")
    fn = functools.partial(
        pl.kernel,
        mesh=mesh,
        out_type=jax.ShapeDtypeStruct((BATCH, 2 * EMB), jnp.float32),
        scratch_types=[
            pltpu.VMEM((K, CHUNK), jnp.int32),
            pltpu.VMEM((K, CHUNK), jnp.int32),
            pltpu.VMEM((K, CHUNK, EMB), jnp.float32),
            pltpu.VMEM((K, CHUNK, EMB), jnp.float32),
            pltpu.SemaphoreType.DMA,
        ],
        compiler_params=pltpu.CompilerParams(use_tc_tiling_on_sc=False),
    )(_gather_body)
    return fn(user_ids, item_ids, user_emb, item_emb)


BS = 2048


def _mlp_body(c_ref, w1_ref, b1_ref, w2t_ref, b2_ref, out_ref):
    x = jnp.dot(c_ref[...], w1_ref[...], preferred_element_type=jnp.float32)
    x = jnp.maximum(x + b1_ref[...], 0.0)
    y = jnp.sum(x * w2t_ref[...], axis=1, keepdims=True)
    out_ref[...] = y + b2_ref[...]


@jax.jit
def _mlp(comb, w1, b1, w2t, b2):
    grid = (BATCH // BS,)
    return pl.pallas_call(
        _mlp_body,
        grid=grid,
        in_specs=[
            pl.BlockSpec((BS, 2 * EMB), lambda g: (g, 0)),
            pl.BlockSpec((2 * EMB, HID), lambda g: (0, 0)),
            pl.BlockSpec((1, HID), lambda g: (0, 0)),
            pl.BlockSpec((1, HID), lambda g: (0, 0)),
            pl.BlockSpec((1, 1), lambda g: (0, 0)),
        ],
        out_specs=pl.BlockSpec((BS, 1), lambda g: (g, 0)),
        out_shape=jax.ShapeDtypeStruct((BATCH, 1), jnp.float32),
    )(comb, w1, b1, w2t, b2)


def kernel(user_ids, item_ids, user_emb, item_emb, W1, b1, W2, b2):
    uids = user_ids.astype(jnp.int32)
    iids = item_ids.astype(jnp.int32)
    comb = _gather(uids, iids, user_emb, item_emb)
    return _mlp(comb, W1, b1.reshape(1, HID), W2.reshape(1, HID),
                b2.reshape(1, 1))
